# baseline (speedup 1.0000x reference)
import jax
import jax.numpy as jnp
import numpy as np
from jax.experimental import pallas as pl
from jax.experimental.pallas import tpu as pltpu

_B, _C, _H, _W, _D = 16, 512, 14, 14, 8192
_NPIX = _B * _H * _W
_HW = _H * _W
_D1, _D2 = 64, 128
_PB = 32
_BT = 4
_NDJ = 8

_CompilerParams = getattr(pltpu, "CompilerParams", None) or pltpu.TPUCompilerParams


def _trig():
    i64 = np.arange(_D1, dtype=np.float64)
    i128 = np.arange(_D2, dtype=np.float64)
    tp = 2.0 * np.pi
    f32 = np.float32
    c128 = np.cos(tp * np.outer(i128, i128) / _D2).astype(f32)
    s128 = np.sin(tp * np.outer(i128, i128) / _D2).astype(f32)
    twi = tp * np.outer(i64, i128) / _D
    twir = np.ascontiguousarray(np.broadcast_to(
        np.cos(twi).astype(f32)[:, None, :], (_D1, _PB, _D2)))
    twii = np.ascontiguousarray(np.broadcast_to(
        np.sin(twi).astype(f32)[:, None, :], (_D1, _PB, _D2)))
    c64i = np.cos(tp * np.outer(i64, i64) / _D1).astype(f32)
    s64i = np.sin(tp * np.outer(i64, i64) / _D1).astype(f32)
    return c128, s128, twir, twii, c64i, s64i


_TRIG = _trig()
_BF16_MATS = [m.astype(jnp.bfloat16) if i in (0, 1) else m
              for i, m in enumerate(_TRIG)]


def _xt_kernel(x_ref, xt_ref):
    slabs = [jnp.swapaxes(x_ref[0, bb], 0, 1).astype(jnp.bfloat16)
             for bb in range(_B)]
    xt_ref[0] = jnp.concatenate(slabs, axis=0)


def _proj_kernel(x_ref, w_ref, b_ref):
    xb = x_ref[0]
    for t in range(4):
        v = jnp.dot(xb, w_ref[0, :, t * 256:(t + 1) * 256],
                    preferred_element_type=jnp.float32)
        v = v.astype(jnp.bfloat16)
        b_ref[0, 2 * t] = v[:, :128]
        b_ref[0, 2 * t + 1] = v[:, 128:]


def _fft_kernel(b_ref, c128, s128, twir, twii, c64i, s64i, out_ref):
    f32 = jnp.float32
    bf16 = jnp.bfloat16
    cm = c128[...]
    sm = s128[...]

    def dot3(t, m):
        return jnp.einsum('kpm,mn->kpn', t, m, preferred_element_type=f32)

    b1re = b_ref[0]
    b1im = b_ref[1]
    b2re = b_ref[2]
    b2im = b_ref[3]
    y1re = dot3(b1re, cm) + dot3(b1im, sm)
    y1im = dot3(b1im, cm) - dot3(b1re, sm)
    y2re = dot3(b2re, cm) + dot3(b2im, sm)
    y2im = dot3(b2im, cm) - dot3(b2re, sm)
    fre = (y1re * y2re - y1im * y2im).astype(bf16)
    fim = (y1re * y2im + y1im * y2re).astype(bf16)
    p1re = dot3(fre, cm) - dot3(fim, sm)
    p1im = dot3(fre, sm) + dot3(fim, cm)
    p2re = (p1re * twir[...] - p1im * twii[...]).astype(bf16)
    p2im = (p1re * twii[...] + p1im * twir[...]).astype(bf16)
    outv = (jnp.einsum('na,apm->npm', c64i[...].astype(bf16), p2re,
                       preferred_element_type=f32)
            - jnp.einsum('na,apm->npm', s64i[...].astype(bf16), p2im,
                         preferred_element_type=f32))
    out_ref[...] = outv


def kernel(x1, x2, S1, S2):
    bf16 = jnp.bfloat16
    f32 = jnp.float32

    xr = jnp.stack([x1.reshape(_B, _C, _HW), x2.reshape(_B, _C, _HW)])
    xt = pl.pallas_call(
        _xt_kernel,
        grid=(2,),
        in_specs=[pl.BlockSpec((1, _B, _C, _HW), lambda i: (i, 0, 0, 0))],
        out_specs=pl.BlockSpec((1, _NPIX, _C), lambda i: (i, 0, 0)),
        out_shape=jax.ShapeDtypeStruct((2, _NPIX, _C), bf16),
        compiler_params=_CompilerParams(
            dimension_semantics=("parallel",),
            vmem_limit_bytes=100 * 1024 * 1024,
        ),
    )(xr)

    k64 = jnp.arange(_D1, dtype=f32)
    lane = jnp.arange(_D2, dtype=jnp.int32)

    def fold(S):
        s = jnp.sum(S, axis=1)
        h = jnp.argmax(jnp.abs(S), axis=1).astype(jnp.int32)
        ph = (2.0 * np.pi / _D) * h.astype(f32)[:, None] * k64[None, :]
        onehot = (lane[None, :] == (h % _D2)[:, None]).astype(f32)
        wre = (jnp.cos(ph) * s[:, None])[:, :, None] * onehot[:, None, :]
        wim = (-jnp.sin(ph) * s[:, None])[:, :, None] * onehot[:, None, :]
        return wre, wim

    w1re, w1im = fold(S1)
    w2re, w2im = fold(S2)
    wq = (jnp.stack([w1re, w1im, w2re, w2im])
          .reshape(4, _C, _D).astype(bf16))

    bq = pl.pallas_call(
        _proj_kernel,
        grid=(4, _NDJ),
        in_specs=[
            pl.BlockSpec((1, _NPIX, _C), lambda w, dj: (w // 2, 0, 0)),
            pl.BlockSpec((1, _C, _D // _NDJ), lambda w, dj: (w, 0, dj)),
        ],
        out_specs=pl.BlockSpec((1, _D1 // _NDJ, _NPIX, _D2),
                               lambda w, dj: (w, dj, 0, 0)),
        out_shape=jax.ShapeDtypeStruct((4, _D1, _NPIX, _D2), bf16),
        compiler_params=_CompilerParams(
            dimension_semantics=("parallel", "parallel"),
            vmem_limit_bytes=100 * 1024 * 1024,
        ),
    )(xt, wq)

    trig = [jnp.asarray(t) for t in _BF16_MATS]
    const_specs = [pl.BlockSpec(t.shape, lambda j, n=t.ndim: (0,) * n)
                   for t in trig]

    outv = pl.pallas_call(
        _fft_kernel,
        grid=(_NPIX // _PB,),
        in_specs=[pl.BlockSpec((4, _D1, _PB, _D2), lambda j: (0, 0, j, 0))]
        + const_specs,
        out_specs=pl.BlockSpec((_D1, _PB, _D2), lambda j: (0, j, 0)),
        out_shape=jax.ShapeDtypeStruct((_D1, _NPIX, _D2), jnp.float32),
        compiler_params=_CompilerParams(
            dimension_semantics=("parallel",),
            vmem_limit_bytes=100 * 1024 * 1024,
        ),
    )(bq, *trig)

    return outv.transpose(1, 0, 2).reshape(_B, _H, _W, _D)


# speedup vs baseline: 1.1890x; 1.1890x over previous
import jax
import jax.numpy as jnp
import numpy as np
from jax.experimental import pallas as pl
from jax.experimental.pallas import tpu as pltpu

_B, _C, _H, _W, _D = 16, 512, 14, 14, 8192
_NPIX = _B * _H * _W
_HW = _H * _W
_D1, _D2 = 64, 128
_PB = 32
_NDJ = 8

_CompilerParams = getattr(pltpu, "CompilerParams", None) or pltpu.TPUCompilerParams


def _trig():
    i64 = np.arange(_D1, dtype=np.float64)
    i128 = np.arange(_D2, dtype=np.float64)
    tp = 2.0 * np.pi
    f32 = np.float32
    bf16 = jnp.bfloat16
    c128 = np.cos(tp * np.outer(i128, i128) / _D2).astype(bf16)
    s128 = np.sin(tp * np.outer(i128, i128) / _D2).astype(bf16)
    twi = tp * np.outer(i64, i128) / _D
    twir = np.ascontiguousarray(np.broadcast_to(
        np.cos(twi).astype(f32)[:, None, :], (_D1, _PB, _D2)))
    twii = np.ascontiguousarray(np.broadcast_to(
        np.sin(twi).astype(f32)[:, None, :], (_D1, _PB, _D2)))
    c64i = np.cos(tp * np.outer(i64, i64) / _D1).astype(bf16)
    s64i = np.sin(tp * np.outer(i64, i64) / _D1).astype(bf16)
    return c128, s128, twir, twii, c64i, s64i


_TRIG = _trig()


def _xt_kernel(x_ref, xt_ref):
    slabs = [jnp.swapaxes(x_ref[0, bb], 0, 1).astype(jnp.bfloat16)
             for bb in range(_B)]
    xt_ref[0] = jnp.concatenate(slabs, axis=0)


def _proj_kernel(x_ref, w_ref, b_ref):
    xb = x_ref[0]
    wb = jnp.concatenate([w_ref[0, :, t, :] for t in range(_D1 // _NDJ)],
                         axis=-1)
    v = jnp.dot(xb, wb, preferred_element_type=jnp.float32)
    v = v.astype(jnp.bfloat16)
    for t in range(_D1 // _NDJ):
        b_ref[0, t] = v[:, t * 128:(t + 1) * 128]


def _fft_kernel(b_ref, c128, s128, twir, twii, c64i, s64i, out_ref):
    f32 = jnp.float32
    bf16 = jnp.bfloat16
    cm = c128[...]
    sm = s128[...]

    def dot3(t, m):
        return jnp.einsum('kpm,mn->kpn', t, m, preferred_element_type=f32)

    y1re = b_ref[0].astype(f32)
    y1im = b_ref[1].astype(f32)
    y2re = b_ref[2].astype(f32)
    y2im = b_ref[3].astype(f32)
    fre = (y1re * y2re - y1im * y2im).astype(bf16)
    fim = (y1re * y2im + y1im * y2re).astype(bf16)
    p1re = dot3(fre, cm) - dot3(fim, sm)
    p1im = dot3(fre, sm) + dot3(fim, cm)
    p2re = (p1re * twir[...] - p1im * twii[...]).astype(bf16)
    p2im = (p1re * twii[...] + p1im * twir[...]).astype(bf16)
    outv = (jnp.einsum('na,apm->npm', c64i[...], p2re,
                       preferred_element_type=f32)
            - jnp.einsum('na,apm->npm', s64i[...], p2im,
                         preferred_element_type=f32))
    for t in range(_D1 // 8):
        out_ref[:, 8 * t:8 * t + 8, :] = jnp.stack(
            [outv[8 * t + s] for s in range(8)], axis=1)


def kernel(x1, x2, S1, S2):
    bf16 = jnp.bfloat16
    f32 = jnp.float32

    xr = jnp.stack([x1.reshape(_B, _C, _HW), x2.reshape(_B, _C, _HW)])
    xt = pl.pallas_call(
        _xt_kernel,
        grid=(2,),
        in_specs=[pl.BlockSpec((1, _B, _C, _HW), lambda i: (i, 0, 0, 0))],
        out_specs=pl.BlockSpec((1, _NPIX, _C), lambda i: (i, 0, 0)),
        out_shape=jax.ShapeDtypeStruct((2, _NPIX, _C), bf16),
        compiler_params=_CompilerParams(
            dimension_semantics=("parallel",),
            vmem_limit_bytes=100 * 1024 * 1024,
        ),
    )(xr)

    k64 = jnp.arange(_D1, dtype=f32)
    k128 = jnp.arange(_D2, dtype=f32)
    iota = jnp.arange(_D, dtype=f32)

    def fold(S):
        s = jnp.sum(S, axis=1)
        h = s * jnp.dot(S, iota)
        pa = (2.0 * np.pi / _D) * h[:, None] * k64[None, :]
        pb = (2.0 * np.pi / _D2) * jnp.mod(h, _D2)[:, None] * k128[None, :]
        ca = jnp.cos(pa) * s[:, None]
        sa = jnp.sin(pa) * s[:, None]
        cb = jnp.cos(pb)
        sb = jnp.sin(pb)
        wre = ca[:, :, None] * cb[:, None, :] - sa[:, :, None] * sb[:, None, :]
        wim = -(sa[:, :, None] * cb[:, None, :] + ca[:, :, None] * sb[:, None, :])
        return wre, wim

    w1re, w1im = fold(S1)
    w2re, w2im = fold(S2)
    wq = jnp.stack([w1re, w1im, w2re, w2im]).astype(bf16)

    bq = pl.pallas_call(
        _proj_kernel,
        grid=(4, _NDJ),
        in_specs=[
            pl.BlockSpec((1, _NPIX, _C), lambda w, dj: (w // 2, 0, 0)),
            pl.BlockSpec((1, _C, _D1 // _NDJ, _D2), lambda w, dj: (w, 0, dj, 0)),
        ],
        out_specs=pl.BlockSpec((1, _D1 // _NDJ, _NPIX, _D2),
                               lambda w, dj: (w, dj, 0, 0)),
        out_shape=jax.ShapeDtypeStruct((4, _D1, _NPIX, _D2), bf16),
        compiler_params=_CompilerParams(
            dimension_semantics=("parallel", "parallel"),
            vmem_limit_bytes=100 * 1024 * 1024,
        ),
    )(xt, wq)

    trig = [jnp.asarray(t) for t in _TRIG]
    const_specs = [pl.BlockSpec(t.shape, lambda j, n=t.ndim: (0,) * n)
                   for t in trig]

    out_nat = pl.pallas_call(
        _fft_kernel,
        grid=(_NPIX // _PB,),
        in_specs=[pl.BlockSpec((4, _D1, _PB, _D2), lambda j: (0, 0, j, 0))]
        + const_specs,
        out_specs=pl.BlockSpec((_PB, _D1, _D2), lambda j: (j, 0, 0)),
        out_shape=jax.ShapeDtypeStruct((_NPIX, _D1, _D2), jnp.float32),
        compiler_params=_CompilerParams(
            dimension_semantics=("parallel",),
            vmem_limit_bytes=100 * 1024 * 1024,
        ),
    )(bq, *trig)

    return out_nat.reshape(_B, _H, _W, _D)


# speedup vs baseline: 1.2205x; 1.0264x over previous
import jax
import jax.numpy as jnp
import numpy as np
from jax.experimental import pallas as pl
from jax.experimental.pallas import tpu as pltpu

_B, _C, _H, _W, _D = 16, 512, 14, 14, 8192
_NPIX = _B * _H * _W
_HW = _H * _W
_D1, _D2 = 64, 128
_PB = 32
_NDJ = 8

_CompilerParams = getattr(pltpu, "CompilerParams", None) or pltpu.TPUCompilerParams


def _trig():
    i64 = np.arange(_D1, dtype=np.float64)
    i128 = np.arange(_D2, dtype=np.float64)
    tp = 2.0 * np.pi
    f32 = np.float32
    bf16 = jnp.bfloat16
    c128 = np.cos(tp * np.outer(i128, i128) / _D2).astype(bf16)
    s128 = np.sin(tp * np.outer(i128, i128) / _D2).astype(bf16)
    twi = tp * np.outer(i64, i128) / _D
    twir = np.ascontiguousarray(np.broadcast_to(
        np.cos(twi).astype(f32)[:, None, :], (_D1, _PB, _D2)))
    twii = np.ascontiguousarray(np.broadcast_to(
        np.sin(twi).astype(f32)[:, None, :], (_D1, _PB, _D2)))
    c64i = np.cos(tp * np.outer(i64, i64) / _D1).astype(bf16)
    s64i = np.sin(tp * np.outer(i64, i64) / _D1).astype(bf16)
    return c128, s128, twir, twii, c64i, s64i


_TRIG = _trig()


def _xt_kernel(x_ref, xt_ref):
    slabs = [jnp.swapaxes(x_ref[0, bb], 0, 1).astype(jnp.bfloat16)
             for bb in range(_B)]
    xt_ref[0] = jnp.concatenate(slabs, axis=0)


def _proj_kernel(x_ref, w_ref, b_ref):
    xb = x_ref[0]
    wb = jnp.concatenate([w_ref[0, :, t, :] for t in range(_D1 // _NDJ)],
                         axis=-1)
    v = jnp.dot(xb, wb, preferred_element_type=jnp.float32)
    v = v.astype(jnp.bfloat16)
    for t in range(_D1 // _NDJ):
        b_ref[0, t] = v[:, t * 128:(t + 1) * 128]


def _fft_kernel(b_ref, c128, s128, twir, twii, c64i, s64i, out_ref):
    f32 = jnp.float32
    bf16 = jnp.bfloat16
    cm = c128[...]
    sm = s128[...]

    def dot3(t, m):
        return jnp.einsum('kpm,mn->kpn', t, m, preferred_element_type=f32)

    y1re = b_ref[0].astype(f32)
    y1im = b_ref[1].astype(f32)
    y2re = b_ref[2].astype(f32)
    y2im = b_ref[3].astype(f32)
    fre = (y1re * y2re - y1im * y2im).astype(bf16)
    fim = (y1re * y2im + y1im * y2re).astype(bf16)
    p1re = dot3(fre, cm) - dot3(fim, sm)
    p1im = dot3(fre, sm) + dot3(fim, cm)
    p2re = (p1re * twir[...] - p1im * twii[...]).astype(bf16)
    p2im = (p1re * twii[...] + p1im * twir[...]).astype(bf16)
    outv = (jnp.einsum('na,apm->npm', c64i[...], p2re,
                       preferred_element_type=f32)
            - jnp.einsum('na,apm->npm', s64i[...], p2im,
                         preferred_element_type=f32))
    for t in range(_D1 // 8):
        out_ref[:, 8 * t:8 * t + 8, :] = jnp.stack(
            [outv[8 * t + s] for s in range(8)], axis=1)


def kernel(x1, x2, S1, S2):
    bf16 = jnp.bfloat16
    f32 = jnp.float32

    xr = jnp.stack([x1.reshape(_B, _C, _HW), x2.reshape(_B, _C, _HW)])
    xt = pl.pallas_call(
        _xt_kernel,
        grid=(2,),
        in_specs=[pl.BlockSpec((1, _B, _C, _HW), lambda i: (i, 0, 0, 0))],
        out_specs=pl.BlockSpec((1, _NPIX, _C), lambda i: (i, 0, 0)),
        out_shape=jax.ShapeDtypeStruct((2, _NPIX, _C), bf16),
        compiler_params=_CompilerParams(
            dimension_semantics=("parallel",),
            vmem_limit_bytes=100 * 1024 * 1024,
        ),
    )(xr)

    k64 = jnp.arange(_D1, dtype=f32)
    k128 = jnp.arange(_D2, dtype=f32)
    iota = jnp.arange(_D, dtype=f32)

    def fold(S):
        s = jnp.sum(S, axis=1)
        h = s * jnp.sum(S * iota[None, :], axis=1)
        pa = (2.0 * np.pi / _D) * h[:, None] * k64[None, :]
        pb = (2.0 * np.pi / _D2) * jnp.mod(h, _D2)[:, None] * k128[None, :]
        ca = jnp.cos(pa) * s[:, None]
        sa = jnp.sin(pa) * s[:, None]
        cb = jnp.cos(pb)
        sb = jnp.sin(pb)
        wre = ca[:, :, None] * cb[:, None, :] - sa[:, :, None] * sb[:, None, :]
        wim = -(sa[:, :, None] * cb[:, None, :] + ca[:, :, None] * sb[:, None, :])
        return wre, wim

    w1re, w1im = fold(S1)
    w2re, w2im = fold(S2)
    wq = jnp.stack([w1re, w1im, w2re, w2im]).astype(bf16)

    bq = pl.pallas_call(
        _proj_kernel,
        grid=(4, _NDJ),
        in_specs=[
            pl.BlockSpec((1, _NPIX, _C), lambda w, dj: (w // 2, 0, 0)),
            pl.BlockSpec((1, _C, _D1 // _NDJ, _D2), lambda w, dj: (w, 0, dj, 0)),
        ],
        out_specs=pl.BlockSpec((1, _D1 // _NDJ, _NPIX, _D2),
                               lambda w, dj: (w, dj, 0, 0)),
        out_shape=jax.ShapeDtypeStruct((4, _D1, _NPIX, _D2), bf16),
        compiler_params=_CompilerParams(
            dimension_semantics=("parallel", "parallel"),
            vmem_limit_bytes=100 * 1024 * 1024,
        ),
    )(xt, wq)

    trig = [jnp.asarray(t) for t in _TRIG]
    const_specs = [pl.BlockSpec(t.shape, lambda j, n=t.ndim: (0,) * n)
                   for t in trig]

    out_nat = pl.pallas_call(
        _fft_kernel,
        grid=(_NPIX // _PB,),
        in_specs=[pl.BlockSpec((4, _D1, _PB, _D2), lambda j: (0, 0, j, 0))]
        + const_specs,
        out_specs=pl.BlockSpec((_PB, _D1, _D2), lambda j: (j, 0, 0)),
        out_shape=jax.ShapeDtypeStruct((_NPIX, _D1, _D2), jnp.float32),
        compiler_params=_CompilerParams(
            dimension_semantics=("parallel",),
            vmem_limit_bytes=100 * 1024 * 1024,
        ),
    )(bq, *trig)

    return out_nat.reshape(_B, _H, _W, _D)


# speedup vs baseline: 1.4249x; 1.1675x over previous
import jax
import jax.numpy as jnp
import numpy as np
from jax.experimental import pallas as pl
from jax.experimental.pallas import tpu as pltpu

_B, _C, _H, _W, _D = 16, 512, 14, 14, 8192
_NPIX = _B * _H * _W
_HW = _H * _W
_D1, _D2 = 64, 128
_PB = 32
_NDJ = 8

_CompilerParams = getattr(pltpu, "CompilerParams", None) or pltpu.TPUCompilerParams


def _trig():
    i64 = np.arange(_D1, dtype=np.float64)
    i128 = np.arange(_D2, dtype=np.float64)
    tp = 2.0 * np.pi
    f32 = np.float32
    bf16 = jnp.bfloat16
    c128 = np.cos(tp * np.outer(i128, i128) / _D2).astype(bf16)
    s128 = np.sin(tp * np.outer(i128, i128) / _D2).astype(bf16)
    twi = tp * np.outer(i64, i128) / _D
    twir = np.ascontiguousarray(np.broadcast_to(
        np.cos(twi).astype(f32)[:, None, :], (_D1, _PB, _D2)))
    twii = np.ascontiguousarray(np.broadcast_to(
        np.sin(twi).astype(f32)[:, None, :], (_D1, _PB, _D2)))
    c64i = np.cos(tp * np.outer(i64, i64) / _D1).astype(f32)
    s64i = np.sin(tp * np.outer(i64, i64) / _D1).astype(f32)
    return c128, s128, twir, twii, c64i, s64i


_TRIG = _trig()


def _xt_kernel(x_ref, xt_ref):
    slabs = [jnp.swapaxes(x_ref[0, bb], 0, 1).astype(jnp.bfloat16)
             for bb in range(_B)]
    xt_ref[0] = jnp.concatenate(slabs, axis=0)


def _proj_kernel(x_ref, w_ref, b_ref):
    xb = x_ref[0]
    wb = jnp.concatenate([w_ref[0, :, t, :] for t in range(_D1 // _NDJ)],
                         axis=-1)
    v = jnp.dot(xb, wb, preferred_element_type=jnp.float32)
    v = v.astype(jnp.bfloat16)
    for t in range(_D1 // _NDJ):
        b_ref[0, t] = v[:, t * 128:(t + 1) * 128]


def _fft_kernel(b_ref, c128, s128, twir, twii, c64i, s64i, out_ref):
    f32 = jnp.float32
    bf16 = jnp.bfloat16
    cm = c128[...]
    sm = s128[...]

    def dot3(t, m):
        return jnp.einsum('kpm,mn->kpn', t, m, preferred_element_type=f32)

    y1re = b_ref[0]
    y1im = b_ref[1]
    y2re = b_ref[2]
    y2im = b_ref[3]
    fre = y1re * y2re - y1im * y2im
    fim = y1re * y2im + y1im * y2re
    p1re = dot3(fre, cm) - dot3(fim, sm)
    p1im = dot3(fre, sm) + dot3(fim, cm)
    p2re = p1re * twir[...] - p1im * twii[...]
    p2im = p1re * twii[...] + p1im * twir[...]
    outv = (jnp.einsum('na,apm->npm', c64i[...], p2re,
                       preferred_element_type=f32)
            - jnp.einsum('na,apm->npm', s64i[...], p2im,
                         preferred_element_type=f32))
    for a in range(_D1):
        out_ref[:, a * _D2:(a + 1) * _D2] = outv[a]


def kernel(x1, x2, S1, S2):
    bf16 = jnp.bfloat16
    f32 = jnp.float32

    xr = jnp.stack([x1.reshape(_B, _C, _HW), x2.reshape(_B, _C, _HW)])
    xt = pl.pallas_call(
        _xt_kernel,
        grid=(2,),
        in_specs=[pl.BlockSpec((1, _B, _C, _HW), lambda i: (i, 0, 0, 0))],
        out_specs=pl.BlockSpec((1, _NPIX, _C), lambda i: (i, 0, 0)),
        out_shape=jax.ShapeDtypeStruct((2, _NPIX, _C), bf16),
        compiler_params=_CompilerParams(
            dimension_semantics=("parallel",),
            vmem_limit_bytes=100 * 1024 * 1024,
        ),
    )(xr)

    k64 = jnp.arange(_D1, dtype=f32)
    k128 = jnp.arange(_D2, dtype=f32)
    iota = jnp.arange(_D, dtype=f32)

    def fold(S):
        s = jnp.sum(S, axis=1)
        h = s * jnp.sum(S * iota[None, :], axis=1)
        pa = (2.0 * np.pi / _D) * h[:, None] * k64[None, :]
        pb = (2.0 * np.pi / _D2) * jnp.mod(h, _D2)[:, None] * k128[None, :]
        ca = jnp.cos(pa) * s[:, None]
        sa = jnp.sin(pa) * s[:, None]
        cb = jnp.cos(pb)
        sb = jnp.sin(pb)
        wre = ca[:, :, None] * cb[:, None, :] - sa[:, :, None] * sb[:, None, :]
        wim = -(sa[:, :, None] * cb[:, None, :] + ca[:, :, None] * sb[:, None, :])
        return wre, wim

    w1re, w1im = fold(S1)
    w2re, w2im = fold(S2)
    wq = jnp.stack([w1re, w1im, w2re, w2im]).astype(bf16)

    bq = pl.pallas_call(
        _proj_kernel,
        grid=(4, _NDJ),
        in_specs=[
            pl.BlockSpec((1, _NPIX, _C), lambda w, dj: (w // 2, 0, 0)),
            pl.BlockSpec((1, _C, _D1 // _NDJ, _D2), lambda w, dj: (w, 0, dj, 0)),
        ],
        out_specs=pl.BlockSpec((1, _D1 // _NDJ, _NPIX, _D2),
                               lambda w, dj: (w, dj, 0, 0)),
        out_shape=jax.ShapeDtypeStruct((4, _D1, _NPIX, _D2), bf16),
        compiler_params=_CompilerParams(
            dimension_semantics=("parallel", "parallel"),
            vmem_limit_bytes=100 * 1024 * 1024,
        ),
    )(xt, wq)

    trig = [jnp.asarray(t) for t in _TRIG]
    const_specs = [pl.BlockSpec(t.shape, lambda j, n=t.ndim: (0,) * n)
                   for t in trig]

    out_nat = pl.pallas_call(
        _fft_kernel,
        grid=(_NPIX // _PB,),
        in_specs=[pl.BlockSpec((4, _D1, _PB, _D2), lambda j: (0, 0, j, 0))]
        + const_specs,
        out_specs=pl.BlockSpec((_PB, _D), lambda j: (j, 0)),
        out_shape=jax.ShapeDtypeStruct((_NPIX, _D), jnp.float32),
        compiler_params=_CompilerParams(
            dimension_semantics=("parallel",),
            vmem_limit_bytes=100 * 1024 * 1024,
        ),
    )(bq, *trig)

    return out_nat.reshape(_B, _H, _W, _D)
